# baseline (speedup 1.0000x reference)
import functools

import jax
import jax.numpy as jnp
from jax import lax
from jax.experimental import pallas as pl
from jax.experimental.pallas import tpu as pltpu
from jax.experimental.pallas import tpu_sc as plsc

_TAU = 2.0 / 3.0
_K = 8
_ZCHUNK = 8192


def _st_zeros():
    mesh = plsc.VectorSubcoreMesh(core_axis_name="c", subcore_axis_name="s")

    @functools.partial(
        pl.kernel,
        mesh=mesh,
        out_type=jax.ShapeDtypeStruct((8, 64, 32768), jnp.float32),
        scratch_types=[
            pltpu.VMEM((_ZCHUNK,), jnp.float32),
            pltpu.SemaphoreType.DMA,
        ],
    )
    def k(out_hbm, zbuf, sem):
        wid = lax.axis_index("s") * 2 + lax.axis_index("c")

        def zero_body(j, carry):
            zbuf[pl.ds(j * 16, 16)] = jnp.zeros((16,), jnp.float32)
            return carry

        lax.fori_loop(0, _ZCHUNK // 16, zero_body, 0)
        i = wid // 4
        r0 = (wid % 4) * 16
        copies = []
        for r in range(16):
            for c in range(32768 // _ZCHUNK):
                copies.append(
                    pltpu.async_copy(
                        zbuf,
                        out_hbm.at[i, r0 + r, pl.ds(c * _ZCHUNK, _ZCHUNK)],
                        sem,
                    )
                )
        for cp in copies:
            cp.wait()

    return k


def _soft_body(logits_ref, gumbel_ref, soft_ref, ework_ref):
    i = pl.program_id(0)
    rb, n = ework_ref.shape

    @pl.when(i == 0)
    def _init():
        z = (logits_ref[...] + gumbel_ref[...]) / _TAU
        ework_ref[...] = jnp.exp(z - jnp.max(z, axis=1, keepdims=True))

    e = ework_ref[...]
    m = jnp.max(e, axis=1, keepdims=True)
    col = jax.lax.broadcasted_iota(jnp.int32, (rb, n), 1)
    idx = jnp.min(jnp.where(e == m, col, n), axis=1, keepdims=True)
    s_inv = 1.0 / jnp.sum(e, axis=1, keepdims=True)
    soft_ref[0] = e * s_inv
    ework_ref[...] = jnp.where(col == idx, 0.0, e)


def _soft_half(logits, gumbel):
    RB, N = logits.shape
    in_spec = pl.BlockSpec((RB, N), lambda i: (0, 0))
    out_spec = pl.BlockSpec((1, RB, N), lambda i: (i, 0, 0))
    (soft,) = pl.pallas_call(
        _soft_body,
        grid=(_K,),
        in_specs=[in_spec, in_spec],
        out_specs=[out_spec],
        out_shape=[jax.ShapeDtypeStruct((_K, RB, N), jnp.float32)],
        scratch_shapes=[pltpu.VMEM((RB, N), jnp.float32)],
        compiler_params=pltpu.CompilerParams(
            dimension_semantics=("arbitrary",),
        ),
    )(logits, gumbel)
    return soft


def kernel(logits, k, gumbel_noise):
    del k
    st = _st_zeros()()
    soft_a = _soft_half(logits[:32], gumbel_noise[:32])
    soft_b = _soft_half(logits[32:], gumbel_noise[32:])
    return st, soft_a, soft_b


# speedup vs baseline: 1.5123x; 1.5123x over previous
import jax
import jax.numpy as jnp
from jax.experimental import pallas as pl
from jax.experimental.pallas import tpu as pltpu

_TAU = 2.0 / 3.0
_K = 8


def _gumbel_topk_body(logits_ref, gumbel_ref, st_ref, soft_ref, ework_ref):
    i = pl.program_id(1)
    rb, n = ework_ref.shape

    @pl.when(i == 0)
    def _init():
        z = (logits_ref[...] + gumbel_ref[...]) / _TAU
        ework_ref[...] = jnp.exp(z - jnp.max(z, axis=1, keepdims=True))

    e = ework_ref[...]
    m = jnp.max(e, axis=1, keepdims=True)
    col = jax.lax.broadcasted_iota(jnp.int32, (rb, n), 1)
    idx = jnp.min(jnp.where(e == m, col, n), axis=1, keepdims=True)
    s_inv = 1.0 / jnp.sum(e, axis=1, keepdims=True)
    soft_ref[0] = e * s_inv
    st_ref[0] = jnp.where(col == idx, 1.0, 0.0)
    ework_ref[...] = jnp.where(col == idx, 0.0, e)


def kernel(logits, k, gumbel_noise):
    del k
    B, N = logits.shape
    RB = 32
    nrb = B // RB
    grid = (nrb, _K)

    in_spec = pl.BlockSpec((RB, N), lambda rb, i: (rb, 0))
    out_spec = pl.BlockSpec((1, RB, N), lambda rb, i: (i, rb, 0))
    st, soft = pl.pallas_call(
        _gumbel_topk_body,
        grid=grid,
        in_specs=[in_spec, in_spec],
        out_specs=[out_spec, out_spec],
        out_shape=[
            jax.ShapeDtypeStruct((_K, B, N), jnp.float32),
            jax.ShapeDtypeStruct((_K, B, N), jnp.float32),
        ],
        scratch_shapes=[pltpu.VMEM((RB, N), jnp.float32)],
        compiler_params=pltpu.CompilerParams(
            dimension_semantics=("arbitrary", "arbitrary"),
        ),
    )(logits, gumbel_noise)
    return st, soft


# speedup vs baseline: 1.5346x; 1.0147x over previous
import jax
import jax.numpy as jnp
from jax.experimental import pallas as pl
from jax.experimental.pallas import tpu as pltpu

_TAU = 2.0 / 3.0
_K = 8


def _gumbel_topk_body(logits_ref, gumbel_ref, st_ref, soft_ref, ework_ref):
    i = pl.program_id(1)
    rb, n = ework_ref.shape

    @pl.when(i == 0)
    def _init():
        z = (logits_ref[...] + gumbel_ref[...]) / _TAU
        ework_ref[...] = jnp.exp(z - jnp.max(z, axis=1, keepdims=True))

    e = ework_ref[...]
    m = jnp.max(e, axis=1, keepdims=True)
    col = jax.lax.broadcasted_iota(jnp.int32, (rb, n), 1)
    idx = jnp.min(jnp.where(e == m, col, n), axis=1, keepdims=True)
    s_inv = 1.0 / jnp.sum(e, axis=1, keepdims=True)
    soft_ref[0] = e * s_inv
    st_ref[0] = jnp.where(col == idx, 1.0, 0.0)
    ework_ref[...] = jnp.where(col == idx, 0.0, e)


def kernel(logits, k, gumbel_noise):
    del k
    B, N = logits.shape
    RB = 32
    nrb = B // RB
    grid = (nrb, _K)

    in_spec = pl.BlockSpec((RB, N), lambda rb, i: (rb, 0))
    out_spec = pl.BlockSpec((1, RB, N), lambda rb, i: (i, rb, 0))
    st, soft = pl.pallas_call(
        _gumbel_topk_body,
        grid=grid,
        in_specs=[in_spec, in_spec],
        out_specs=[out_spec, out_spec],
        out_shape=[
            jax.ShapeDtypeStruct((_K, B, N), jnp.float32),
            jax.ShapeDtypeStruct((_K, B, N), jnp.float32),
        ],
        scratch_shapes=[pltpu.VMEM((RB, N), jnp.float32)],
        compiler_params=pltpu.CompilerParams(
            dimension_semantics=("parallel", "arbitrary"),
        ),
    )(logits, gumbel_noise)
    return st, soft
